# Initial kernel scaffold; baseline (speedup 1.0000x reference)
import jax
import jax.numpy as jnp
from jax.experimental import pallas as pl


def kernel(pos, positions, spatial_table, temporal_table):
    raise NotImplementedError("write your pallas kernel here")


# speedup vs baseline: 1.0075x; 1.0075x over previous
import functools

import jax
import jax.numpy as jnp
from jax import lax
from jax.experimental import pallas as pl
from jax.experimental.pallas import tpu as pltpu
from jax.experimental.pallas import tpu_sc as plsc


def _nn_body(q_ref, q2_ref, p_ref, p2_ref, out_ref, bv_ref, bi_ref, *, tn):
    j = pl.program_id(0)
    qb = q_ref[...]
    pb = p_ref[...]
    dots = jnp.dot(qb, pb, preferred_element_type=jnp.float32)
    d = q2_ref[...] + p2_ref[...] - 2.0 * dots
    m = jnp.min(d, axis=1, keepdims=True)
    lane = lax.broadcasted_iota(jnp.int32, d.shape, 1)
    li = jnp.min(jnp.where(d == m, lane, tn), axis=1, keepdims=True)
    gi = li + j * tn

    @pl.when(j == 0)
    def _():
        bv_ref[...] = m
        bi_ref[...] = gi

    @pl.when(j > 0)
    def _():
        better = m < bv_ref[...]
        bv_ref[...] = jnp.where(better, m, bv_ref[...])
        bi_ref[...] = jnp.where(better, gi, bi_ref[...])

    @pl.when(j == pl.num_programs(0) - 1)
    def _():
        out_ref[...] = bi_ref[...]


def _nn_indices(q, q2, pos_t, p2p, tn):
    qn = q.shape[0]
    n_tiles = pos_t.shape[1] // tn
    out = pl.pallas_call(
        functools.partial(_nn_body, tn=tn),
        grid=(n_tiles,),
        in_specs=[
            pl.BlockSpec((qn, 3), lambda j: (0, 0)),
            pl.BlockSpec((qn, 1), lambda j: (0, 0)),
            pl.BlockSpec((3, tn), lambda j: (0, j)),
            pl.BlockSpec((1, tn), lambda j: (0, j)),
        ],
        out_specs=pl.BlockSpec((qn, 1), lambda j: (0, 0)),
        out_shape=jax.ShapeDtypeStruct((qn, 1), jnp.int32),
        scratch_shapes=[
            pltpu.VMEM((qn, 1), jnp.float32),
            pltpu.VMEM((qn, 1), jnp.int32),
        ],
    )(q, q2, pos_t, p2p)
    return out.reshape(qn)


def _sc_gather_rows(table, idx):
    info = plsc.get_sparse_core_info()
    nc, ns = info.num_cores, info.num_subcores
    nw = nc * ns
    qn = idx.shape[0]
    e = table.shape[1]
    b_per_w = qn // nw
    mesh = plsc.VectorSubcoreMesh(core_axis_name="c", subcore_axis_name="s")

    @functools.partial(
        pl.kernel,
        mesh=mesh,
        out_type=jax.ShapeDtypeStruct((qn, e), jnp.float32),
        scratch_types=[
            pltpu.VMEM((b_per_w,), jnp.int32),
            pltpu.VMEM((b_per_w, e), jnp.float32),
            pltpu.SemaphoreType.DMA,
        ],
    )
    def gather_k(table_hbm, idx_hbm, out_hbm, idx_v, rows_v, sem):
        wid = lax.axis_index("s") * nc + lax.axis_index("c")
        base = wid * b_per_w
        pltpu.sync_copy(idx_hbm.at[pl.ds(base, b_per_w)], idx_v)
        pltpu.async_copy(table_hbm.at[idx_v], rows_v, sem).wait()
        pltpu.sync_copy(rows_v, out_hbm.at[pl.ds(base, b_per_w)])

    return gather_k(table, idx)


def _expand_body(g_ref, t_ref, out_ref):
    g = g_ref[...]
    t = t_ref[...]
    out_ref[...] = g[:, None, :] + t[None, :, :]


def _expand_add(gathered, temporal, qb):
    qn, e = gathered.shape
    t = temporal.shape[0]
    return pl.pallas_call(
        _expand_body,
        grid=(qn // qb,),
        in_specs=[
            pl.BlockSpec((qb, e), lambda i: (i, 0)),
            pl.BlockSpec((t, e), lambda i: (0, 0)),
        ],
        out_specs=pl.BlockSpec((qb, t, e), lambda i: (i, 0, 0)),
        out_shape=jax.ShapeDtypeStruct((qn, t, e), jnp.float32),
    )(gathered, temporal)


def kernel(pos, positions, spatial_table, temporal_table):
    b, c, _ = pos.shape
    n, e = spatial_table.shape
    t = temporal_table.shape[0]
    qn = b * c

    q = pos.reshape(qn, 3)
    q2 = jnp.sum(pos * pos, axis=-1, keepdims=True).reshape(qn, 1)
    p2 = jnp.sum(positions * positions, axis=-1)

    tn = 1024
    n_tiles = -(-n // tn)
    n_pad = n_tiles * tn
    pos_t = jnp.pad(positions, ((0, n_pad - n), (0, 0))).T
    p2p = jnp.pad(p2, (0, n_pad - n), constant_values=1e30).reshape(1, n_pad)

    idx = _nn_indices(q, q2, pos_t, p2p, tn)
    gathered = _sc_gather_rows(spatial_table, idx)
    out = _expand_add(gathered, temporal_table, 128)
    return out.reshape(b, c * t, e)
